# baseline (speedup 1.0000x reference)
import functools

import jax
import jax.numpy as jnp
from jax import lax
from jax.experimental import pallas as pl
from jax.experimental.pallas import tpu as pltpu
from jax.experimental.pallas import tpu_sc as plsc

L = 16
ROW = 8192
NCHUNK = ROW // L
NROWS = 64 * 32
NWORK = 32
ROWS_PER_W = NROWS // NWORK
CAND_MAX = 2048
NEG = -1e30


def _sc_body(x_hbm, out_hbm, rowbuf, outbuf, candi, candv):
    wid = lax.axis_index("s") * 2 + lax.axis_index("c")
    iota = lax.iota(jnp.int32, L)
    zeros = jnp.zeros((L,), jnp.float32)

    def row_body(r, _):
        row = wid * ROWS_PER_W + r
        pltpu.sync_copy(x_hbm.at[row], rowbuf)

        @plsc.parallel_loop(
            0, NCHUNK, unroll=8, carry=jnp.full((L,), NEG, jnp.float32)
        )
        def mx(c, acc):
            v = rowbuf[pl.ds(c * L, L)]
            outbuf[pl.ds(c * L, L)] = zeros
            return jnp.maximum(acc, v)

        bound = jnp.full((L,), jnp.max(mx) - jnp.float32(1.0), jnp.float32)

        @pl.loop(0, NCHUNK, init_carry=jnp.zeros((L,), jnp.int32), unroll=8)
        def off_vec(c, off):
            v = rowbuf[pl.ds(c * L, L)]
            m = v > bound
            idx = c * L + iota
            pos = off + plsc.cumsum(jnp.where(m, 1, 0)) - 1
            plsc.store_scatter(candi, [pos], idx, mask=m)
            cnt = plsc.all_reduce_population_count(m)
            return jnp.minimum(off + cnt, CAND_MAX)

        k = off_vec[0]
        nk = (k + L - 1) // L

        def gbody(j, _):
            ok = j * L + iota < k
            idxs = jnp.where(ok, candi[pl.ds(j * L, L)], 0)
            v = plsc.load_gather(rowbuf, [idxs])
            candv[pl.ds(j * L, L)] = jnp.where(ok, v, NEG)
            return 0

        lax.fori_loop(0, nk, gbody, 0)

        def cond(carry):
            i, _, changed = carry
            return changed & (i < 300)

        def step(carry):
            i, tau, _ = carry

            def ibody(j, acc):
                s, c = acc
                v = candv[pl.ds(j * L, L)]
                m = v > tau
                return s + jnp.where(m, v, 0.0), c + jnp.where(m, 1, 0)

            s, c = lax.fori_loop(0, nk, ibody, (zeros, jnp.zeros((L,), jnp.int32)))
            csum = jnp.maximum(jnp.sum(c), 1).astype(jnp.float32)
            ssum = jnp.sum(s)
            tau_new = (jnp.full((L,), ssum) - jnp.float32(1.0)) / jnp.full((L,), csum)
            return i + 1, tau_new, jnp.any(tau_new != tau)

        _, tau, _ = lax.while_loop(cond, step, (0, bound, True))

        def sbody(j, _):
            ok = j * L + iota < k
            idxs = jnp.where(ok, candi[pl.ds(j * L, L)], 0)
            w = jnp.maximum(candv[pl.ds(j * L, L)] - tau, 0.0)
            plsc.store_scatter(outbuf, [idxs], w, mask=ok)
            return 0

        lax.fori_loop(0, nk, sbody, 0)

        pltpu.sync_copy(outbuf, out_hbm.at[row])
        return 0

    lax.fori_loop(0, ROWS_PER_W, row_body, 0)


@jax.jit
def _sparsemax_sc(x):
    f = pl.kernel(
        _sc_body,
        out_type=jax.ShapeDtypeStruct((NROWS, ROW), jnp.float32),
        mesh=plsc.VectorSubcoreMesh(core_axis_name="c", subcore_axis_name="s"),
        scratch_types=[
            pltpu.VMEM((ROW,), jnp.float32),
            pltpu.VMEM((ROW,), jnp.float32),
            pltpu.VMEM((CAND_MAX + L,), jnp.int32),
            pltpu.VMEM((CAND_MAX + L,), jnp.float32),
        ],
        compiler_params=pltpu.CompilerParams(needs_layout_passes=False),
    )
    return f(x)


def kernel(inputs, mask):
    del mask
    x = inputs.reshape(NROWS, ROW)
    return _sparsemax_sc(x).reshape(inputs.shape)


# speedup vs baseline: 1.2732x; 1.2732x over previous
import jax
import jax.numpy as jnp
from jax import lax
from jax.experimental import pallas as pl
from jax.experimental.pallas import tpu as pltpu
from jax.experimental.pallas import tpu_sc as plsc

L = 16
ROW = 8192
NCHUNK = ROW // L
NROWS = 64 * 32
NWORK = 32
ROWS_PER_W = NROWS // NWORK
CAND_MAX = 2048
NEG = -1e30


def _sc_body(x_hbm, out_hbm, rowbuf0, rowbuf1, outbuf0, outbuf1, candi, candv, insem, outsem):
    rowbufs = (rowbuf0, rowbuf1)
    outbufs = (outbuf0, outbuf1)
    wid = lax.axis_index("s") * 2 + lax.axis_index("c")
    base = wid * ROWS_PER_W
    iota = lax.iota(jnp.int32, L)
    zeros = jnp.zeros((L,), jnp.float32)

    def compute_row(rb, ob):
        @plsc.parallel_loop(
            0, NCHUNK, unroll=8, carry=jnp.full((L,), NEG, jnp.float32)
        )
        def mx(c, acc):
            v = rb[pl.ds(c * L, L)]
            ob[pl.ds(c * L, L)] = zeros
            return jnp.maximum(acc, v)

        bound = jnp.full((L,), jnp.max(mx) - jnp.float32(1.0), jnp.float32)

        @pl.loop(0, NCHUNK, init_carry=0, unroll=8)
        def k(c, off):
            v = rb[pl.ds(c * L, L)]
            m = v > bound
            idx = c * L + iota
            plsc.store_compressed(candi.at[pl.ds(off, L)], idx, mask=m)
            cnt = jnp.sum(jnp.where(m, 1, 0))
            return jnp.minimum(off + cnt, CAND_MAX)

        nk = (k + L - 1) // L

        def gbody(j, _):
            ok = j * L + iota < k
            idxs = jnp.where(ok, candi[pl.ds(j * L, L)], 0)
            v = plsc.load_gather(rb, [idxs])
            candv[pl.ds(j * L, L)] = jnp.where(ok, v, jnp.float32(NEG))
            return 0

        lax.fori_loop(0, nk, gbody, 0)

        def cond(carry):
            i, _, changed = carry
            return changed & (i < 300)

        def step(carry):
            i, tau, _ = carry

            def ibody(j, acc):
                s, c = acc
                v = candv[pl.ds(j * L, L)]
                m = v > tau
                return s + jnp.where(m, v, 0.0), c + jnp.where(m, 1, 0)

            s, c = lax.fori_loop(0, nk, ibody, (zeros, jnp.zeros((L,), jnp.int32)))
            csum = jnp.maximum(jnp.sum(c), 1).astype(jnp.float32)
            ssum = jnp.sum(s)
            tau_new = (jnp.full((L,), ssum) - jnp.float32(1.0)) / jnp.full((L,), csum)
            return i + 1, tau_new, jnp.any(tau_new != tau)

        _, tau, _ = lax.while_loop(cond, step, (0, bound, True))

        def sbody(j, _):
            ok = j * L + iota < k
            idxs = jnp.where(ok, candi[pl.ds(j * L, L)], 0)
            w = jnp.maximum(candv[pl.ds(j * L, L)] - tau, 0.0)
            plsc.store_scatter(ob, [idxs], w, mask=ok)
            return 0

        lax.fori_loop(0, nk, sbody, 0)

    def in_copy(r, b):
        return pltpu.make_async_copy(x_hbm.at[base + r], rowbufs[b], insem.at[b])

    def out_copy(r, b):
        return pltpu.make_async_copy(outbufs[b], out_hbm.at[base + r], outsem.at[b])

    in_copy(0, 0).start()

    @pl.loop(0, ROWS_PER_W, step=2)
    def _(r0):
        for b in range(2):
            r = r0 + b
            nb = 1 - b

            @pl.when(r + 1 < ROWS_PER_W)
            def _():
                in_copy(r + 1, nb).start()

            in_copy(r, b).wait()

            @pl.when(r >= 2)
            def _():
                out_copy(r - 2, b).wait()

            compute_row(rowbufs[b], outbufs[b])
            out_copy(r, b).start()

    for b in range(2):
        out_copy(ROWS_PER_W - 2 + b, b).wait()


@jax.jit
def _sparsemax_sc(x):
    f = pl.kernel(
        _sc_body,
        out_type=jax.ShapeDtypeStruct((NROWS, ROW), jnp.float32),
        mesh=plsc.VectorSubcoreMesh(core_axis_name="c", subcore_axis_name="s"),
        scratch_types=[
            pltpu.VMEM((ROW,), jnp.float32),
            pltpu.VMEM((ROW,), jnp.float32),
            pltpu.VMEM((ROW,), jnp.float32),
            pltpu.VMEM((ROW,), jnp.float32),
            pltpu.VMEM((CAND_MAX + L,), jnp.int32),
            pltpu.VMEM((CAND_MAX + L,), jnp.float32),
            pltpu.SemaphoreType.DMA((2,)),
            pltpu.SemaphoreType.DMA((2,)),
        ],
        compiler_params=pltpu.CompilerParams(needs_layout_passes=False),
    )
    return f(x)


def kernel(inputs, mask):
    del mask
    x = inputs.reshape(NROWS, ROW)
    return _sparsemax_sc(x).reshape(inputs.shape)


# speedup vs baseline: 1.7204x; 1.3512x over previous
import jax
import jax.numpy as jnp
from jax import lax
from jax.experimental import pallas as pl
from jax.experimental.pallas import tpu as pltpu
from jax.experimental.pallas import tpu_sc as plsc

L = 16
ROW = 8192
NCHUNK = ROW // L
NGROUP = NCHUNK // L
NSAMP = 128
NROWS = 64 * 32
NWORK = 32
ROWS_PER_W = NROWS // NWORK
CAND_MAX = 2048
NEG = -1e30


def _sc_body(
    x_hbm,
    out_hbm,
    rowbuf0,
    rowbuf1,
    outbuf0,
    outbuf1,
    cntbuf,
    chunklist,
    chunkoff,
    candv,
    insem,
    outsem,
):
    rowbufs = (rowbuf0, rowbuf1)
    outbufs = (outbuf0, outbuf1)
    wid = lax.axis_index("s") * 2 + lax.axis_index("c")
    base = wid * ROWS_PER_W
    iota = lax.iota(jnp.int32, L)
    zeros = jnp.zeros((L,), jnp.float32)
    izeros = jnp.zeros((L,), jnp.int32)
    onehot = [iota == j for j in range(L)]

    def compute_row(rb, ob):
        @plsc.parallel_loop(0, NSAMP, unroll=8, carry=jnp.full((L,), NEG, jnp.float32))
        def smx(c, acc):
            return jnp.maximum(acc, rb[pl.ds(c * L, L)])

        bound = jnp.full((L,), jnp.max(smx) - jnp.float32(1.0), jnp.float32)

        @pl.loop(0, NGROUP)
        def _(g):
            acc = izeros
            for j in range(L):
                c = g * L + j
                v = rb[pl.ds(c * L, L)]
                ob[pl.ds(c * L, L)] = zeros
                cnt = plsc.all_reduce_population_count(v > bound)
                acc = jnp.where(onehot[j], cnt, acc)
            cntbuf[pl.ds(g * L, L)] = acc

        def gbody(g, carry):
            off_f, off_splat = carry
            cnt16 = cntbuf[pl.ds(g * L, L)]
            m_g = cnt16 > 0
            prefix = plsc.cumsum(cnt16)
            offs = jnp.minimum(off_splat + prefix - cnt16, CAND_MAX)
            ids = g * L + iota
            plsc.store_compressed(chunklist.at[pl.ds(off_f, L)], ids, mask=m_g)
            plsc.store_compressed(chunkoff.at[pl.ds(off_f, L)], offs, mask=m_g)
            nf = jnp.sum(jnp.where(m_g, 1, 0))
            total = lax.squeeze(lax.slice(prefix, (L - 1,), (L,)), (0,))
            return off_f + nf, off_splat + jnp.full((L,), total)

        nflag, off_splat = lax.fori_loop(0, NGROUP, gbody, (0, izeros))
        k = jnp.minimum(lax.squeeze(lax.slice(off_splat, (0,), (1,)), (0,)), CAND_MAX)

        def _lane0(vec):
            return lax.squeeze(lax.slice(vec, (0,), (1,)), (0,))

        def fbody(i, _):
            cid = _lane0(chunklist[pl.ds(i, L)])
            o = _lane0(chunkoff[pl.ds(i, L)])
            v = rb[pl.ds(cid * L, L)]
            plsc.store_compressed(candv.at[pl.ds(o, L)], v, mask=v > bound)
            return 0

        lax.fori_loop(0, nflag, fbody, 0)
        candv[pl.ds(k, L)] = jnp.full((L,), NEG, jnp.float32)
        nk = (k + L - 1) // L

        def cond(carry):
            i, _, changed = carry
            return changed & (i < 300)

        def step(carry):
            i, tau, _ = carry

            def ibody(j, acc):
                s, c = acc
                v = candv[pl.ds(j * L, L)]
                m = v > tau
                return s + jnp.where(m, v, 0.0), c + jnp.where(m, 1, 0)

            s, c = lax.fori_loop(0, nk, ibody, (zeros, izeros))
            csum = jnp.maximum(jnp.sum(c), 1).astype(jnp.float32)
            ssum = jnp.sum(s)
            tau_new = (jnp.full((L,), ssum) - jnp.float32(1.0)) / jnp.full((L,), csum)
            return i + 1, tau_new, jnp.any(tau_new != tau)

        _, tau, _ = lax.while_loop(cond, step, (0, bound, True))

        def obody(i, _):
            cid = _lane0(chunklist[pl.ds(i, L)])
            v = rb[pl.ds(cid * L, L)]
            ob[pl.ds(cid * L, L)] = jnp.maximum(v - tau, 0.0)
            return 0

        lax.fori_loop(0, nflag, obody, 0)

    def in_copy(r, b):
        return pltpu.make_async_copy(x_hbm.at[base + r], rowbufs[b], insem.at[b])

    def out_copy(r, b):
        return pltpu.make_async_copy(outbufs[b], out_hbm.at[base + r], outsem.at[b])

    in_copy(0, 0).start()

    @pl.loop(0, ROWS_PER_W, step=2)
    def _(r0):
        for b in range(2):
            r = r0 + b
            nb = 1 - b

            @pl.when(r + 1 < ROWS_PER_W)
            def _():
                in_copy(r + 1, nb).start()

            in_copy(r, b).wait()

            @pl.when(r >= 2)
            def _():
                out_copy(r - 2, b).wait()

            compute_row(rowbufs[b], outbufs[b])
            out_copy(r, b).start()

    for b in range(2):
        out_copy(ROWS_PER_W - 2 + b, b).wait()


@jax.jit
def _sparsemax_sc(x):
    f = pl.kernel(
        _sc_body,
        out_type=jax.ShapeDtypeStruct((NROWS, ROW), jnp.float32),
        mesh=plsc.VectorSubcoreMesh(core_axis_name="c", subcore_axis_name="s"),
        scratch_types=[
            pltpu.VMEM((ROW,), jnp.float32),
            pltpu.VMEM((ROW,), jnp.float32),
            pltpu.VMEM((ROW,), jnp.float32),
            pltpu.VMEM((ROW,), jnp.float32),
            pltpu.VMEM((NCHUNK,), jnp.int32),
            pltpu.VMEM((NCHUNK + L,), jnp.int32),
            pltpu.VMEM((NCHUNK + L,), jnp.int32),
            pltpu.VMEM((CAND_MAX + L,), jnp.float32),
            pltpu.SemaphoreType.DMA((2,)),
            pltpu.SemaphoreType.DMA((2,)),
        ],
        compiler_params=pltpu.CompilerParams(needs_layout_passes=False),
    )
    return f(x)


def kernel(inputs, mask):
    del mask
    x = inputs.reshape(NROWS, ROW)
    return _sparsemax_sc(x).reshape(inputs.shape)


# speedup vs baseline: 2.4452x; 1.4213x over previous
import jax
import jax.numpy as jnp
from jax import lax
from jax.experimental import pallas as pl
from jax.experimental.pallas import tpu as pltpu
from jax.experimental.pallas import tpu_sc as plsc

L = 16
ROW = 8192
NCHUNK = ROW // L
NGROUP = NCHUNK // L
NSAMP = 128
NROWS = 64 * 32
NWORK = 32
ROWS_PER_W = NROWS // NWORK
CAND_MAX = 2048
NEG = -1e30


def _sc_body(
    x_hbm,
    out_hbm,
    rowbuf0,
    rowbuf1,
    outbuf0,
    outbuf1,
    cntbuf,
    chunklist,
    chunkoff,
    candv,
    candidx,
    insem,
    outsem,
):
    rowbufs = (rowbuf0, rowbuf1)
    outbufs = (outbuf0, outbuf1)
    wid = lax.axis_index("s") * 2 + lax.axis_index("c")
    base = wid * ROWS_PER_W
    iota = lax.iota(jnp.int32, L)
    zeros = jnp.zeros((L,), jnp.float32)
    izeros = jnp.zeros((L,), jnp.int32)
    onehot = [iota == j for j in range(L)]

    def _lane0(vec):
        return lax.squeeze(lax.slice(vec, (0,), (1,)), (0,))

    def compute_row(rb, ob):
        @plsc.parallel_loop(0, NSAMP, unroll=8, carry=jnp.full((L,), NEG, jnp.float32))
        def smx(c, acc):
            return jnp.maximum(acc, rb[pl.ds(c * L, L)])

        bound = jnp.full((L,), jnp.max(smx) - jnp.float32(1.0), jnp.float32)

        @pl.loop(0, NGROUP)
        def _(g):
            acc = izeros
            for j in range(L):
                c = g * L + j
                v = rb[pl.ds(c * L, L)]
                ob[pl.ds(c * L, L)] = zeros
                cnt = plsc.all_reduce_population_count(v > bound)
                acc = jnp.where(onehot[j], cnt, acc)
            cntbuf[pl.ds(g * L, L)] = acc

        def gbody(g, carry):
            off_f, off_splat = carry
            cnt16 = cntbuf[pl.ds(g * L, L)]
            m_g = cnt16 > 0
            prefix = plsc.cumsum(cnt16)
            offs = jnp.minimum(off_splat + prefix - cnt16, CAND_MAX)
            ids = g * L + iota
            plsc.store_compressed(chunklist.at[pl.ds(off_f, L)], ids, mask=m_g)
            plsc.store_compressed(chunkoff.at[pl.ds(off_f, L)], offs, mask=m_g)
            nf = _lane0(plsc.all_reduce_population_count(m_g))
            total = lax.squeeze(lax.slice(prefix, (L - 1,), (L,)), (0,))
            return off_f + nf, off_splat + jnp.full((L,), total)

        nflag, off_splat = lax.fori_loop(0, NGROUP, gbody, (0, izeros))
        k = jnp.minimum(_lane0(off_splat), CAND_MAX)

        def fbody(i, _):
            cid = _lane0(chunklist[pl.ds(i, L)])
            o = _lane0(chunkoff[pl.ds(i, L)])
            v = rb[pl.ds(cid * L, L)]
            m = v > bound
            plsc.store_compressed(candv.at[pl.ds(o, L)], v, mask=m)
            plsc.store_compressed(candidx.at[pl.ds(o, L)], cid * L + iota, mask=m)
            return 0

        lax.fori_loop(0, nflag, fbody, 0)
        candv[pl.ds(k, L)] = jnp.full((L,), NEG, jnp.float32)
        nk = (k + L - 1) // L

        def cond(carry):
            i, _, changed = carry
            return changed & (i < 300)

        def step(carry):
            i, tau, _ = carry

            def ibody(j, acc):
                s, c = acc
                v = candv[pl.ds(j * L, L)]
                m = v > tau
                return s + jnp.where(m, v, 0.0), c + jnp.where(m, 1, 0)

            s, c = lax.fori_loop(0, nk, ibody, (zeros, izeros))
            csum = jnp.maximum(jnp.sum(c), 1).astype(jnp.float32)
            ssum = jnp.sum(s)
            tau_new = (jnp.full((L,), ssum) - jnp.float32(1.0)) / jnp.full((L,), csum)
            changed = _lane0(plsc.all_reduce_population_count(tau_new != tau)) > 0
            return i + 1, tau_new, changed

        _, tau, _ = lax.while_loop(cond, step, (0, bound, True))

        def obody(j, _):
            ok = j * L + iota < k
            idxs = jnp.where(ok, candidx[pl.ds(j * L, L)], 0)
            w = jnp.maximum(candv[pl.ds(j * L, L)] - tau, 0.0)
            plsc.store_scatter(ob, [idxs], w, mask=ok)
            return 0

        lax.fori_loop(0, nk, obody, 0)

    def in_copy(r, b):
        return pltpu.make_async_copy(x_hbm.at[base + r], rowbufs[b], insem.at[b])

    def out_copy(r, b):
        return pltpu.make_async_copy(outbufs[b], out_hbm.at[base + r], outsem.at[b])

    in_copy(0, 0).start()

    @pl.loop(0, ROWS_PER_W, step=2)
    def _(r0):
        for b in range(2):
            r = r0 + b
            nb = 1 - b

            @pl.when(r + 1 < ROWS_PER_W)
            def _():
                in_copy(r + 1, nb).start()

            in_copy(r, b).wait()

            @pl.when(r >= 2)
            def _():
                out_copy(r - 2, b).wait()

            compute_row(rowbufs[b], outbufs[b])
            out_copy(r, b).start()

    for b in range(2):
        out_copy(ROWS_PER_W - 2 + b, b).wait()


@jax.jit
def _sparsemax_sc(x):
    f = pl.kernel(
        _sc_body,
        out_type=jax.ShapeDtypeStruct((NROWS, ROW), jnp.float32),
        mesh=plsc.VectorSubcoreMesh(core_axis_name="c", subcore_axis_name="s"),
        scratch_types=[
            pltpu.VMEM((ROW,), jnp.float32),
            pltpu.VMEM((ROW,), jnp.float32),
            pltpu.VMEM((ROW,), jnp.float32),
            pltpu.VMEM((ROW,), jnp.float32),
            pltpu.VMEM((NCHUNK,), jnp.int32),
            pltpu.VMEM((NCHUNK + L,), jnp.int32),
            pltpu.VMEM((NCHUNK + L,), jnp.int32),
            pltpu.VMEM((CAND_MAX + L,), jnp.float32),
            pltpu.VMEM((CAND_MAX + L,), jnp.int32),
            pltpu.SemaphoreType.DMA((2,)),
            pltpu.SemaphoreType.DMA((2,)),
        ],
        compiler_params=pltpu.CompilerParams(needs_layout_passes=False),
    )
    return f(x)


def kernel(inputs, mask):
    del mask
    x = inputs.reshape(NROWS, ROW)
    return _sparsemax_sc(x).reshape(inputs.shape)
